# baseline (speedup 1.0000x reference)
import functools

import jax
import jax.numpy as jnp
from jax import lax
from jax.experimental import pallas as pl
from jax.experimental.pallas import tpu as pltpu
from jax.experimental.pallas import tpu_sc as plsc

BATCH = 16384
EMBED = 64
NC = 2
NS = 16
NW = NC * NS
B_PER_W = BATCH // NW
CHUNK = 128
NCHUNK = B_PER_W // CHUNK
PAIR = 2 * EMBED


def _sc_gather_body(uid_hbm, iid_hbm, utab_hbm, itab_hbm,
                    uout_hbm, iout_hbm,
                    uidx_v, iidx_v, rows_v, sem):
    wid = lax.axis_index("s") * NC + lax.axis_index("c")
    base = wid * B_PER_W
    pltpu.sync_copy(uid_hbm.at[wid], uidx_v)
    pltpu.sync_copy(iid_hbm.at[wid], iidx_v)
    copies = []
    for j in range(NCHUNK):
        copies.append(pltpu.async_copy(
            utab_hbm.at[uidx_v.at[j]], rows_v.at[pl.ds(j * CHUNK, CHUNK)], sem))
    for c in copies:
        c.wait()
    pltpu.sync_copy(rows_v, uout_hbm.at[pl.ds(base, B_PER_W)])
    copies = []
    for j in range(NCHUNK):
        copies.append(pltpu.async_copy(
            itab_hbm.at[iidx_v.at[j]], rows_v.at[pl.ds(j * CHUNK, CHUNK)], sem))
    for c in copies:
        c.wait()
    pltpu.sync_copy(rows_v, iout_hbm.at[pl.ds(base, B_PER_W)])


_sc_gather = functools.partial(
    pl.kernel,
    out_type=[jax.ShapeDtypeStruct((BATCH, PAIR), jnp.float32),
              jax.ShapeDtypeStruct((BATCH, PAIR), jnp.float32)],
    mesh=plsc.VectorSubcoreMesh(core_axis_name="c", subcore_axis_name="s"),
    scratch_types=[
        pltpu.VMEM((NCHUNK, CHUNK), jnp.int32),
        pltpu.VMEM((NCHUNK, CHUNK), jnp.int32),
        pltpu.VMEM((B_PER_W, PAIR), jnp.float32),
        pltpu.SemaphoreType.DMA,
    ],
)(_sc_gather_body)


def _mlp_body(u_ref, i_ref, up_ref, ip_ref, w1u_ref, w1i_ref, b1_ref,
              w2_ref, b2_ref, w3_ref, b3_ref, out_ref):
    u = jnp.where(up_ref[...] > 0, u_ref[:, EMBED:], u_ref[:, :EMBED])
    i = jnp.where(ip_ref[...] > 0, i_ref[:, EMBED:], i_ref[:, :EMBED])
    h = jnp.dot(u, w1u_ref[...], preferred_element_type=jnp.float32)
    h = h + jnp.dot(i, w1i_ref[...], preferred_element_type=jnp.float32)
    h = jnp.maximum(h + b1_ref[...], 0.0)
    h2 = jnp.dot(h, w2_ref[...], preferred_element_type=jnp.float32)
    h2 = jnp.maximum(h2 + b2_ref[...], 0.0)
    out = jnp.sum(h2 * w3_ref[...], axis=1, keepdims=True) + b3_ref[...]
    out_ref[...] = out


def _mlp(u, i, upar, ipar, w1u, w1i, b1, w2, b2, w3_row, b3):
    blk = 2048
    return pl.pallas_call(
        _mlp_body,
        grid=(BATCH // blk,),
        in_specs=[
            pl.BlockSpec((blk, PAIR), lambda g: (g, 0)),
            pl.BlockSpec((blk, PAIR), lambda g: (g, 0)),
            pl.BlockSpec((blk, 1), lambda g: (g, 0)),
            pl.BlockSpec((blk, 1), lambda g: (g, 0)),
            pl.BlockSpec((EMBED, 128), lambda g: (0, 0)),
            pl.BlockSpec((EMBED, 128), lambda g: (0, 0)),
            pl.BlockSpec((1, 128), lambda g: (0, 0)),
            pl.BlockSpec((128, 64), lambda g: (0, 0)),
            pl.BlockSpec((1, 64), lambda g: (0, 0)),
            pl.BlockSpec((1, 64), lambda g: (0, 0)),
            pl.BlockSpec((1, 1), lambda g: (0, 0)),
        ],
        out_specs=pl.BlockSpec((blk, 1), lambda g: (g, 0)),
        out_shape=jax.ShapeDtypeStruct((BATCH, 1), jnp.float32),
    )(u, i, upar, ipar, w1u, w1i, b1, w2, b2, w3_row, b3)


def kernel(user_ids, item_ids, user_table, item_table, W1, b1, W2, b2, W3, b3):
    uid = user_ids.astype(jnp.int32)
    iid = item_ids.astype(jnp.int32)
    upair = (uid >> 1).reshape(NW, NCHUNK, CHUNK)
    ipair = (iid >> 1).reshape(NW, NCHUNK, CHUNK)
    upar = (uid & 1).reshape(BATCH, 1)
    ipar = (iid & 1).reshape(BATCH, 1)
    utab2 = user_table.reshape(-1, PAIR)
    itab2 = item_table.reshape(-1, PAIR)
    u, i = _sc_gather(upair, ipair, utab2, itab2)
    return _mlp(u, i, upar, ipar, W1[:EMBED], W1[EMBED:], b1.reshape(1, 128),
                W2, b2.reshape(1, 64), W3.reshape(1, 64), b3.reshape(1, 1))


# speedup vs baseline: 1.2027x; 1.2027x over previous
import functools

import jax
import jax.numpy as jnp
from jax import lax
from jax.experimental import pallas as pl
from jax.experimental.pallas import tpu as pltpu
from jax.experimental.pallas import tpu_sc as plsc

BATCH = 16384
EMBED = 64
NROWS = 1000000
NPAIR = NROWS // 2
NC = 2
NS = 16
NW = NC * NS
B_PER_W = BATCH // NW
CHUNK = 128
NCHUNK = B_PER_W // CHUNK
PAIR = 2 * EMBED
RELAYOUT_C = 1024
RELAYOUT_G = 489
HALF = RELAYOUT_G * RELAYOUT_C


def _relayout_body(lo_ref, hi_ref, out_ref):
    t_lo = jnp.transpose(lo_ref[...])
    t_hi = jnp.transpose(hi_ref[...])
    out_ref[...] = jnp.concatenate([t_lo, t_hi], axis=1)


def _relayout(tabT):
    return pl.pallas_call(
        _relayout_body,
        grid=(RELAYOUT_G,),
        in_specs=[
            pl.BlockSpec((EMBED, RELAYOUT_C), lambda c: (0, c)),
            pl.BlockSpec(
                (EMBED, RELAYOUT_C),
                lambda c: (0, jnp.minimum(c + RELAYOUT_G, NROWS // RELAYOUT_C))),
        ],
        out_specs=pl.BlockSpec((RELAYOUT_C, PAIR), lambda c: (c, 0)),
        out_shape=jax.ShapeDtypeStruct((HALF, PAIR), jnp.float32),
    )(tabT, tabT)


def _sc_gather_body(uid_hbm, iid_hbm, utab_hbm, itab_hbm,
                    uout_hbm, iout_hbm,
                    uidx_v, iidx_v, rows_v, sem):
    wid = lax.axis_index("s") * NC + lax.axis_index("c")
    base = wid * B_PER_W
    pltpu.sync_copy(uid_hbm.at[wid], uidx_v)
    pltpu.sync_copy(iid_hbm.at[wid], iidx_v)
    copies = []
    for j in range(NCHUNK):
        copies.append(pltpu.async_copy(
            utab_hbm.at[uidx_v.at[j]], rows_v.at[pl.ds(j * CHUNK, CHUNK)], sem))
    for c in copies:
        c.wait()
    pltpu.sync_copy(rows_v, uout_hbm.at[pl.ds(base, B_PER_W)])
    copies = []
    for j in range(NCHUNK):
        copies.append(pltpu.async_copy(
            itab_hbm.at[iidx_v.at[j]], rows_v.at[pl.ds(j * CHUNK, CHUNK)], sem))
    for c in copies:
        c.wait()
    pltpu.sync_copy(rows_v, iout_hbm.at[pl.ds(base, B_PER_W)])


_sc_gather = functools.partial(
    pl.kernel,
    out_type=[jax.ShapeDtypeStruct((BATCH, PAIR), jnp.float32),
              jax.ShapeDtypeStruct((BATCH, PAIR), jnp.float32)],
    mesh=plsc.VectorSubcoreMesh(core_axis_name="c", subcore_axis_name="s"),
    scratch_types=[
        pltpu.VMEM((NCHUNK, CHUNK), jnp.int32),
        pltpu.VMEM((NCHUNK, CHUNK), jnp.int32),
        pltpu.VMEM((B_PER_W, PAIR), jnp.float32),
        pltpu.SemaphoreType.DMA,
    ],
)(_sc_gather_body)


def _mlp_body(u_ref, i_ref, up_ref, ip_ref, w1u_ref, w1i_ref, b1_ref,
              w2_ref, b2_ref, w3_ref, b3_ref, out_ref):
    u = jnp.where(up_ref[...] > 0, u_ref[:, EMBED:], u_ref[:, :EMBED])
    i = jnp.where(ip_ref[...] > 0, i_ref[:, EMBED:], i_ref[:, :EMBED])
    h = jnp.dot(u, w1u_ref[...], preferred_element_type=jnp.float32)
    h = h + jnp.dot(i, w1i_ref[...], preferred_element_type=jnp.float32)
    h = jnp.maximum(h + b1_ref[...], 0.0)
    h2 = jnp.dot(h, w2_ref[...], preferred_element_type=jnp.float32)
    h2 = jnp.maximum(h2 + b2_ref[...], 0.0)
    out = jnp.sum(h2 * w3_ref[...], axis=1, keepdims=True) + b3_ref[...]
    out_ref[...] = out


def _mlp(u, i, upar, ipar, w1u, w1i, b1, w2, b2, w3_row, b3):
    blk = 2048
    return pl.pallas_call(
        _mlp_body,
        grid=(BATCH // blk,),
        in_specs=[
            pl.BlockSpec((blk, PAIR), lambda g: (g, 0)),
            pl.BlockSpec((blk, PAIR), lambda g: (g, 0)),
            pl.BlockSpec((blk, 1), lambda g: (g, 0)),
            pl.BlockSpec((blk, 1), lambda g: (g, 0)),
            pl.BlockSpec((EMBED, 128), lambda g: (0, 0)),
            pl.BlockSpec((EMBED, 128), lambda g: (0, 0)),
            pl.BlockSpec((1, 128), lambda g: (0, 0)),
            pl.BlockSpec((128, 64), lambda g: (0, 0)),
            pl.BlockSpec((1, 64), lambda g: (0, 0)),
            pl.BlockSpec((1, 64), lambda g: (0, 0)),
            pl.BlockSpec((1, 1), lambda g: (0, 0)),
        ],
        out_specs=pl.BlockSpec((blk, 1), lambda g: (g, 0)),
        out_shape=jax.ShapeDtypeStruct((BATCH, 1), jnp.float32),
    )(u, i, upar, ipar, w1u, w1i, b1, w2, b2, w3_row, b3)


def kernel(user_ids, item_ids, user_table, item_table, W1, b1, W2, b2, W3, b3):
    uid = user_ids.astype(jnp.int32)
    iid = item_ids.astype(jnp.int32)
    upair = jnp.where(uid >= HALF, uid - HALF, uid).reshape(NW, NCHUNK, CHUNK)
    ipair = jnp.where(iid >= HALF, iid - HALF, iid).reshape(NW, NCHUNK, CHUNK)
    upar = (uid >= HALF).astype(jnp.int32).reshape(BATCH, 1)
    ipar = (iid >= HALF).astype(jnp.int32).reshape(BATCH, 1)
    utab2 = _relayout(user_table.T)
    itab2 = _relayout(item_table.T)
    u, i = _sc_gather(upair, ipair, utab2, itab2)
    return _mlp(u, i, upar, ipar, W1[:EMBED], W1[EMBED:], b1.reshape(1, 128),
                W2, b2.reshape(1, 64), W3.reshape(1, 64), b3.reshape(1, 1))


# speedup vs baseline: 1.6011x; 1.3312x over previous
import functools

import jax
import jax.numpy as jnp
from jax import lax
from jax.experimental import pallas as pl
from jax.experimental.pallas import tpu as pltpu
from jax.experimental.pallas import tpu_sc as plsc

BATCH = 16384
EMBED = 64
NROWS = 1000000
NPAIR = NROWS // 2
NC = 2
NS = 16
NW = NC * NS
B_PER_W = BATCH // NW
CHUNK = 128
NCHUNK = B_PER_W // CHUNK
PAIR = 2 * EMBED
RELAYOUT_C = 2048
RELAYOUT_G = 245
HALF = RELAYOUT_G * RELAYOUT_C


def _relayout_body(lo_ref, hi_ref, eye_ref, out_ref):
    dn = (((0,), (0,)), ((), ()))
    t_lo = lax.dot_general(lo_ref[...], eye_ref[...], dn,
                           preferred_element_type=jnp.float32)
    t_hi = lax.dot_general(hi_ref[...], eye_ref[...], dn,
                           preferred_element_type=jnp.float32)
    out_ref[...] = jnp.concatenate([t_lo, t_hi], axis=1)


def _relayout(tabT):
    return pl.pallas_call(
        _relayout_body,
        grid=(RELAYOUT_G,),
        in_specs=[
            pl.BlockSpec((EMBED, RELAYOUT_C), lambda c: (0, c)),
            pl.BlockSpec(
                (EMBED, RELAYOUT_C),
                lambda c: (0, jnp.minimum(c + RELAYOUT_G, NROWS // RELAYOUT_C))),
            pl.BlockSpec((EMBED, EMBED), lambda c: (0, 0)),
        ],
        out_specs=pl.BlockSpec((RELAYOUT_C, PAIR), lambda c: (c, 0)),
        out_shape=jax.ShapeDtypeStruct((HALF, PAIR), jnp.float32),
    )(tabT, tabT, jnp.eye(EMBED, dtype=jnp.float32))


def _sc_gather_body(uid_hbm, iid_hbm, utab_hbm, itab_hbm,
                    uout_hbm, iout_hbm,
                    uidx_v, iidx_v, rows_v, sem):
    wid = lax.axis_index("s") * NC + lax.axis_index("c")
    base = wid * B_PER_W
    pltpu.sync_copy(uid_hbm.at[wid], uidx_v)
    pltpu.sync_copy(iid_hbm.at[wid], iidx_v)
    copies = []
    for j in range(NCHUNK):
        copies.append(pltpu.async_copy(
            utab_hbm.at[uidx_v.at[j]], rows_v.at[pl.ds(j * CHUNK, CHUNK)], sem))
    for c in copies:
        c.wait()
    pltpu.sync_copy(rows_v, uout_hbm.at[pl.ds(base, B_PER_W)])
    copies = []
    for j in range(NCHUNK):
        copies.append(pltpu.async_copy(
            itab_hbm.at[iidx_v.at[j]], rows_v.at[pl.ds(j * CHUNK, CHUNK)], sem))
    for c in copies:
        c.wait()
    pltpu.sync_copy(rows_v, iout_hbm.at[pl.ds(base, B_PER_W)])


_sc_gather = functools.partial(
    pl.kernel,
    out_type=[jax.ShapeDtypeStruct((BATCH, PAIR), jnp.float32),
              jax.ShapeDtypeStruct((BATCH, PAIR), jnp.float32)],
    mesh=plsc.VectorSubcoreMesh(core_axis_name="c", subcore_axis_name="s"),
    scratch_types=[
        pltpu.VMEM((NCHUNK, CHUNK), jnp.int32),
        pltpu.VMEM((NCHUNK, CHUNK), jnp.int32),
        pltpu.VMEM((B_PER_W, PAIR), jnp.float32),
        pltpu.SemaphoreType.DMA,
    ],
)(_sc_gather_body)


def _mlp_body(u_ref, i_ref, up_ref, ip_ref, w1u_ref, w1i_ref, b1_ref,
              w2_ref, b2_ref, w3_ref, b3_ref, out_ref):
    u = jnp.where(up_ref[...] > 0, u_ref[:, EMBED:], u_ref[:, :EMBED])
    i = jnp.where(ip_ref[...] > 0, i_ref[:, EMBED:], i_ref[:, :EMBED])
    h = jnp.dot(u, w1u_ref[...], preferred_element_type=jnp.float32)
    h = h + jnp.dot(i, w1i_ref[...], preferred_element_type=jnp.float32)
    h = jnp.maximum(h + b1_ref[...], 0.0)
    h2 = jnp.dot(h, w2_ref[...], preferred_element_type=jnp.float32)
    h2 = jnp.maximum(h2 + b2_ref[...], 0.0)
    out = jnp.sum(h2 * w3_ref[...], axis=1, keepdims=True) + b3_ref[...]
    out_ref[...] = out


def _mlp(u, i, upar, ipar, w1u, w1i, b1, w2, b2, w3_row, b3):
    blk = 2048
    return pl.pallas_call(
        _mlp_body,
        grid=(BATCH // blk,),
        in_specs=[
            pl.BlockSpec((blk, PAIR), lambda g: (g, 0)),
            pl.BlockSpec((blk, PAIR), lambda g: (g, 0)),
            pl.BlockSpec((blk, 1), lambda g: (g, 0)),
            pl.BlockSpec((blk, 1), lambda g: (g, 0)),
            pl.BlockSpec((EMBED, 128), lambda g: (0, 0)),
            pl.BlockSpec((EMBED, 128), lambda g: (0, 0)),
            pl.BlockSpec((1, 128), lambda g: (0, 0)),
            pl.BlockSpec((128, 64), lambda g: (0, 0)),
            pl.BlockSpec((1, 64), lambda g: (0, 0)),
            pl.BlockSpec((1, 64), lambda g: (0, 0)),
            pl.BlockSpec((1, 1), lambda g: (0, 0)),
        ],
        out_specs=pl.BlockSpec((blk, 1), lambda g: (g, 0)),
        out_shape=jax.ShapeDtypeStruct((BATCH, 1), jnp.float32),
    )(u, i, upar, ipar, w1u, w1i, b1, w2, b2, w3_row, b3)


def kernel(user_ids, item_ids, user_table, item_table, W1, b1, W2, b2, W3, b3):
    uid = user_ids.astype(jnp.int32)
    iid = item_ids.astype(jnp.int32)
    upair = jnp.where(uid >= HALF, uid - HALF, uid).reshape(NW, NCHUNK, CHUNK)
    ipair = jnp.where(iid >= HALF, iid - HALF, iid).reshape(NW, NCHUNK, CHUNK)
    upar = (uid >= HALF).astype(jnp.int32).reshape(BATCH, 1)
    ipar = (iid >= HALF).astype(jnp.int32).reshape(BATCH, 1)
    utab2 = _relayout(user_table.T)
    itab2 = _relayout(item_table.T)
    u, i = _sc_gather(upair, ipair, utab2, itab2)
    return _mlp(u, i, upar, ipar, W1[:EMBED], W1[EMBED:], b1.reshape(1, 128),
                W2, b2.reshape(1, 64), W3.reshape(1, 64), b3.reshape(1, 1))


# speedup vs baseline: 1.9838x; 1.2391x over previous
import functools

import jax
import jax.numpy as jnp
from jax import lax
from jax.experimental import pallas as pl
from jax.experimental.pallas import tpu as pltpu
from jax.experimental.pallas import tpu_sc as plsc

BATCH = 16384
EMBED = 64
NROWS = 1000000
NPAIR = NROWS // 2
NC = 2
NS = 16
NW = NC * NS
B_PER_W = BATCH // NW
CHUNK = 128
NCHUNK = B_PER_W // CHUNK
PAIR = 2 * EMBED
RELAYOUT_C = 4096
RELAYOUT_G = 123
HALF = RELAYOUT_G * RELAYOUT_C


def _relayout_body(lo_ref, hi_ref, eye_ref, out_ref):
    dn = (((0,), (0,)), ((), ()))
    t_lo = lax.dot_general(lo_ref[...], eye_ref[...], dn,
                           preferred_element_type=jnp.float32)
    t_hi = lax.dot_general(hi_ref[...], eye_ref[...], dn,
                           preferred_element_type=jnp.float32)
    out_ref[...] = jnp.concatenate([t_lo, t_hi], axis=1)


def _relayout(tabT):
    return pl.pallas_call(
        _relayout_body,
        grid=(RELAYOUT_G,),
        in_specs=[
            pl.BlockSpec((EMBED, RELAYOUT_C), lambda c: (0, c)),
            pl.BlockSpec(
                (EMBED, RELAYOUT_C),
                lambda c: (0, jnp.minimum(c + RELAYOUT_G, NROWS // RELAYOUT_C))),
            pl.BlockSpec((EMBED, EMBED), lambda c: (0, 0)),
        ],
        out_specs=pl.BlockSpec((RELAYOUT_C, PAIR), lambda c: (c, 0)),
        out_shape=jax.ShapeDtypeStruct((HALF, PAIR), jnp.float32),
        compiler_params=pltpu.CompilerParams(fuse_transposed_lhs_in_matmul=True),
    )(tabT, tabT, jnp.eye(EMBED, dtype=jnp.float32))


def _sc_gather_body(id_hbm, tab_hbm, out_hbm, idx_v, rows_v, sem):
    wid = lax.axis_index("s") * NC + lax.axis_index("c")
    base = wid * B_PER_W
    pltpu.sync_copy(id_hbm.at[wid], idx_v)
    copies = []
    for j in range(NCHUNK):
        copies.append(pltpu.async_copy(
            tab_hbm.at[idx_v.at[j]], rows_v.at[pl.ds(j * CHUNK, CHUNK)], sem))
    for c in copies:
        c.wait()
    pltpu.sync_copy(rows_v, out_hbm.at[pl.ds(base, B_PER_W)])


_sc_gather = functools.partial(
    pl.kernel,
    out_type=jax.ShapeDtypeStruct((BATCH, PAIR), jnp.float32),
    mesh=plsc.VectorSubcoreMesh(core_axis_name="c", subcore_axis_name="s"),
    scratch_types=[
        pltpu.VMEM((NCHUNK, CHUNK), jnp.int32),
        pltpu.VMEM((B_PER_W, PAIR), jnp.float32),
        pltpu.SemaphoreType.DMA,
    ],
)(_sc_gather_body)


def _mlp_body(u_ref, i_ref, up_ref, ip_ref, w1u_ref, w1i_ref, b1_ref,
              w2_ref, b2_ref, w3_ref, b3_ref, out_ref):
    u = jnp.where(up_ref[...] > 0, u_ref[:, EMBED:], u_ref[:, :EMBED])
    i = jnp.where(ip_ref[...] > 0, i_ref[:, EMBED:], i_ref[:, :EMBED])
    h = jnp.dot(u, w1u_ref[...], preferred_element_type=jnp.float32)
    h = h + jnp.dot(i, w1i_ref[...], preferred_element_type=jnp.float32)
    h = jnp.maximum(h + b1_ref[...], 0.0)
    h2 = jnp.dot(h, w2_ref[...], preferred_element_type=jnp.float32)
    h2 = jnp.maximum(h2 + b2_ref[...], 0.0)
    out = jnp.sum(h2 * w3_ref[...], axis=1, keepdims=True) + b3_ref[...]
    out_ref[...] = out


def _mlp(u, i, upar, ipar, w1u, w1i, b1, w2, b2, w3_row, b3):
    blk = 2048
    return pl.pallas_call(
        _mlp_body,
        grid=(BATCH // blk,),
        in_specs=[
            pl.BlockSpec((blk, PAIR), lambda g: (g, 0)),
            pl.BlockSpec((blk, PAIR), lambda g: (g, 0)),
            pl.BlockSpec((blk, 1), lambda g: (g, 0)),
            pl.BlockSpec((blk, 1), lambda g: (g, 0)),
            pl.BlockSpec((EMBED, 128), lambda g: (0, 0)),
            pl.BlockSpec((EMBED, 128), lambda g: (0, 0)),
            pl.BlockSpec((1, 128), lambda g: (0, 0)),
            pl.BlockSpec((128, 64), lambda g: (0, 0)),
            pl.BlockSpec((1, 64), lambda g: (0, 0)),
            pl.BlockSpec((1, 64), lambda g: (0, 0)),
            pl.BlockSpec((1, 1), lambda g: (0, 0)),
        ],
        out_specs=pl.BlockSpec((blk, 1), lambda g: (g, 0)),
        out_shape=jax.ShapeDtypeStruct((BATCH, 1), jnp.float32),
    )(u, i, upar, ipar, w1u, w1i, b1, w2, b2, w3_row, b3)


def kernel(user_ids, item_ids, user_table, item_table, W1, b1, W2, b2, W3, b3):
    uid = user_ids.astype(jnp.int32)
    iid = item_ids.astype(jnp.int32)
    upair = jnp.where(uid >= HALF, uid - HALF, uid).reshape(NW, NCHUNK, CHUNK)
    ipair = jnp.where(iid >= HALF, iid - HALF, iid).reshape(NW, NCHUNK, CHUNK)
    upar = (uid >= HALF).astype(jnp.int32).reshape(BATCH, 1)
    ipar = (iid >= HALF).astype(jnp.int32).reshape(BATCH, 1)
    utab2 = _relayout(user_table.T)
    u = _sc_gather(upair, utab2)
    itab2 = _relayout(item_table.T)
    i = _sc_gather(ipair, itab2)
    return _mlp(u, i, upar, ipar, W1[:EMBED], W1[EMBED:], b1.reshape(1, 128),
                W2, b2.reshape(1, 64), W3.reshape(1, 64), b3.reshape(1, 1))


# speedup vs baseline: 2.2492x; 1.1338x over previous
import functools

import jax
import jax.numpy as jnp
from jax import lax
from jax.experimental import pallas as pl
from jax.experimental.pallas import tpu as pltpu
from jax.experimental.pallas import tpu_sc as plsc

BATCH = 16384
EMBED = 64
NROWS = 1000000
NPAIR = NROWS // 2
NC = 2
NS = 16
NW = NC * NS
B_PER_W = BATCH // NW
CHUNK = 128
NCHUNK = B_PER_W // CHUNK
PAIR = 2 * EMBED
RELAYOUT_C = 8192
RELAYOUT_G = 62
HALF = RELAYOUT_G * RELAYOUT_C


def _relayout_body(lo_ref, hi_ref, eye_ref, out_ref):
    dn = (((0,), (0,)), ((), ()))
    t_lo = lax.dot_general(lo_ref[...], eye_ref[...], dn,
                           preferred_element_type=jnp.float32)
    t_hi = lax.dot_general(hi_ref[...], eye_ref[...], dn,
                           preferred_element_type=jnp.float32)
    out_ref[...] = jnp.concatenate([t_lo, t_hi], axis=1)


def _relayout(tabT):
    return pl.pallas_call(
        _relayout_body,
        grid=(RELAYOUT_G,),
        in_specs=[
            pl.BlockSpec((EMBED, RELAYOUT_C), lambda c: (0, c)),
            pl.BlockSpec(
                (EMBED, RELAYOUT_C),
                lambda c: (0, jnp.minimum(c + RELAYOUT_G, NROWS // RELAYOUT_C))),
            pl.BlockSpec((EMBED, EMBED), lambda c: (0, 0)),
        ],
        out_specs=pl.BlockSpec((RELAYOUT_C, PAIR), lambda c: (c, 0)),
        out_shape=jax.ShapeDtypeStruct((HALF, PAIR), jnp.float32),
        compiler_params=pltpu.CompilerParams(fuse_transposed_lhs_in_matmul=True),
    )(tabT, tabT, jnp.eye(EMBED, dtype=jnp.float32))


def _sc_gather_body(id_hbm, tab_hbm, out_hbm, idx_v, rows_v, sem):
    wid = lax.axis_index("s") * NC + lax.axis_index("c")
    base = wid * B_PER_W
    pltpu.sync_copy(id_hbm.at[wid], idx_v)
    copies = []
    for j in range(NCHUNK):
        copies.append(pltpu.async_copy(
            tab_hbm.at[idx_v.at[j]], rows_v.at[pl.ds(j * CHUNK, CHUNK)], sem))
    for c in copies:
        c.wait()
    pltpu.sync_copy(rows_v, out_hbm.at[pl.ds(base, B_PER_W)])


_sc_gather = functools.partial(
    pl.kernel,
    out_type=jax.ShapeDtypeStruct((BATCH, PAIR), jnp.float32),
    mesh=plsc.VectorSubcoreMesh(core_axis_name="c", subcore_axis_name="s"),
    scratch_types=[
        pltpu.VMEM((NCHUNK, CHUNK), jnp.int32),
        pltpu.VMEM((B_PER_W, PAIR), jnp.float32),
        pltpu.SemaphoreType.DMA,
    ],
)(_sc_gather_body)


def _mlp_body(u_ref, i_ref, up_ref, ip_ref, w1u_ref, w1i_ref, b1_ref,
              w2_ref, b2_ref, w3_ref, b3_ref, out_ref):
    u = jnp.where(up_ref[...] > 0, u_ref[:, EMBED:], u_ref[:, :EMBED])
    i = jnp.where(ip_ref[...] > 0, i_ref[:, EMBED:], i_ref[:, :EMBED])
    h = jnp.dot(u, w1u_ref[...], preferred_element_type=jnp.float32)
    h = h + jnp.dot(i, w1i_ref[...], preferred_element_type=jnp.float32)
    h = jnp.maximum(h + b1_ref[...], 0.0)
    h2 = jnp.dot(h, w2_ref[...], preferred_element_type=jnp.float32)
    h2 = jnp.maximum(h2 + b2_ref[...], 0.0)
    out = jnp.sum(h2 * w3_ref[...], axis=1, keepdims=True) + b3_ref[...]
    out_ref[...] = out


def _mlp(u, i, upar, ipar, w1u, w1i, b1, w2, b2, w3_row, b3):
    blk = 2048
    return pl.pallas_call(
        _mlp_body,
        grid=(BATCH // blk,),
        in_specs=[
            pl.BlockSpec((blk, PAIR), lambda g: (g, 0)),
            pl.BlockSpec((blk, PAIR), lambda g: (g, 0)),
            pl.BlockSpec((blk, 1), lambda g: (g, 0)),
            pl.BlockSpec((blk, 1), lambda g: (g, 0)),
            pl.BlockSpec((EMBED, 128), lambda g: (0, 0)),
            pl.BlockSpec((EMBED, 128), lambda g: (0, 0)),
            pl.BlockSpec((1, 128), lambda g: (0, 0)),
            pl.BlockSpec((128, 64), lambda g: (0, 0)),
            pl.BlockSpec((1, 64), lambda g: (0, 0)),
            pl.BlockSpec((1, 64), lambda g: (0, 0)),
            pl.BlockSpec((1, 1), lambda g: (0, 0)),
        ],
        out_specs=pl.BlockSpec((blk, 1), lambda g: (g, 0)),
        out_shape=jax.ShapeDtypeStruct((BATCH, 1), jnp.float32),
    )(u, i, upar, ipar, w1u, w1i, b1, w2, b2, w3_row, b3)


def kernel(user_ids, item_ids, user_table, item_table, W1, b1, W2, b2, W3, b3):
    uid = user_ids.astype(jnp.int32)
    iid = item_ids.astype(jnp.int32)
    upair = jnp.where(uid >= HALF, uid - HALF, uid).reshape(NW, NCHUNK, CHUNK)
    ipair = jnp.where(iid >= HALF, iid - HALF, iid).reshape(NW, NCHUNK, CHUNK)
    upar = (uid >= HALF).astype(jnp.int32).reshape(BATCH, 1)
    ipar = (iid >= HALF).astype(jnp.int32).reshape(BATCH, 1)
    utab2 = _relayout(user_table.T)
    u = _sc_gather(upair, utab2)
    itab2 = _relayout(item_table.T)
    i = _sc_gather(ipair, itab2)
    return _mlp(u, i, upar, ipar, W1[:EMBED], W1[EMBED:], b1.reshape(1, 128),
                W2, b2.reshape(1, 64), W3.reshape(1, 64), b3.reshape(1, 1))


# speedup vs baseline: 2.3742x; 1.0556x over previous
import functools

import jax
import jax.numpy as jnp
from jax import lax
from jax.experimental import pallas as pl
from jax.experimental.pallas import tpu as pltpu
from jax.experimental.pallas import tpu_sc as plsc

BATCH = 16384
EMBED = 64
NROWS = 1000000
NPAIR = NROWS // 2
NC = 2
NS = 16
NW = NC * NS
B_PER_W = BATCH // NW
CHUNK = 128
NCHUNK = B_PER_W // CHUNK
PAIR = 2 * EMBED
RELAYOUT_C = 16384
RELAYOUT_G = 31
HALF = RELAYOUT_G * RELAYOUT_C


def _relayout_body(lo_ref, hi_ref, eye_ref, out_ref):
    dn = (((0,), (0,)), ((), ()))
    t_lo = lax.dot_general(lo_ref[...], eye_ref[...], dn,
                           preferred_element_type=jnp.float32)
    t_hi = lax.dot_general(hi_ref[...], eye_ref[...], dn,
                           preferred_element_type=jnp.float32)
    out_ref[...] = jnp.concatenate([t_lo, t_hi], axis=1)


def _relayout(tabT):
    return pl.pallas_call(
        _relayout_body,
        grid=(RELAYOUT_G,),
        in_specs=[
            pl.BlockSpec((EMBED, RELAYOUT_C), lambda c: (0, c)),
            pl.BlockSpec(
                (EMBED, RELAYOUT_C),
                lambda c: (0, jnp.minimum(c + RELAYOUT_G, NROWS // RELAYOUT_C))),
            pl.BlockSpec((EMBED, EMBED), lambda c: (0, 0)),
        ],
        out_specs=pl.BlockSpec((RELAYOUT_C, PAIR), lambda c: (c, 0)),
        out_shape=jax.ShapeDtypeStruct((HALF, PAIR), jnp.float32),
        compiler_params=pltpu.CompilerParams(fuse_transposed_lhs_in_matmul=True),
    )(tabT, tabT, jnp.eye(EMBED, dtype=jnp.float32))


def _sc_gather_body(id_hbm, tab_hbm, out_hbm, idx_v, rows_v, sem):
    wid = lax.axis_index("s") * NC + lax.axis_index("c")
    base = wid * B_PER_W
    pltpu.sync_copy(id_hbm.at[wid], idx_v)
    copies = []
    for j in range(NCHUNK):
        copies.append(pltpu.async_copy(
            tab_hbm.at[idx_v.at[j]], rows_v.at[pl.ds(j * CHUNK, CHUNK)], sem))
    for c in copies:
        c.wait()
    pltpu.sync_copy(rows_v, out_hbm.at[pl.ds(base, B_PER_W)])


_sc_gather = functools.partial(
    pl.kernel,
    out_type=jax.ShapeDtypeStruct((BATCH, PAIR), jnp.float32),
    mesh=plsc.VectorSubcoreMesh(core_axis_name="c", subcore_axis_name="s"),
    scratch_types=[
        pltpu.VMEM((NCHUNK, CHUNK), jnp.int32),
        pltpu.VMEM((B_PER_W, PAIR), jnp.float32),
        pltpu.SemaphoreType.DMA,
    ],
)(_sc_gather_body)


def _mlp_body(u_ref, i_ref, up_ref, ip_ref, w1u_ref, w1i_ref, b1_ref,
              w2_ref, b2_ref, w3_ref, b3_ref, out_ref):
    u = jnp.where(up_ref[...] > 0, u_ref[:, EMBED:], u_ref[:, :EMBED])
    i = jnp.where(ip_ref[...] > 0, i_ref[:, EMBED:], i_ref[:, :EMBED])
    h = jnp.dot(u, w1u_ref[...], preferred_element_type=jnp.float32)
    h = h + jnp.dot(i, w1i_ref[...], preferred_element_type=jnp.float32)
    h = jnp.maximum(h + b1_ref[...], 0.0)
    h2 = jnp.dot(h, w2_ref[...], preferred_element_type=jnp.float32)
    h2 = jnp.maximum(h2 + b2_ref[...], 0.0)
    out = jnp.sum(h2 * w3_ref[...], axis=1, keepdims=True) + b3_ref[...]
    out_ref[...] = out


def _mlp(u, i, upar, ipar, w1u, w1i, b1, w2, b2, w3_row, b3):
    blk = 2048
    return pl.pallas_call(
        _mlp_body,
        grid=(BATCH // blk,),
        in_specs=[
            pl.BlockSpec((blk, PAIR), lambda g: (g, 0)),
            pl.BlockSpec((blk, PAIR), lambda g: (g, 0)),
            pl.BlockSpec((blk, 1), lambda g: (g, 0)),
            pl.BlockSpec((blk, 1), lambda g: (g, 0)),
            pl.BlockSpec((EMBED, 128), lambda g: (0, 0)),
            pl.BlockSpec((EMBED, 128), lambda g: (0, 0)),
            pl.BlockSpec((1, 128), lambda g: (0, 0)),
            pl.BlockSpec((128, 64), lambda g: (0, 0)),
            pl.BlockSpec((1, 64), lambda g: (0, 0)),
            pl.BlockSpec((1, 64), lambda g: (0, 0)),
            pl.BlockSpec((1, 1), lambda g: (0, 0)),
        ],
        out_specs=pl.BlockSpec((blk, 1), lambda g: (g, 0)),
        out_shape=jax.ShapeDtypeStruct((BATCH, 1), jnp.float32),
    )(u, i, upar, ipar, w1u, w1i, b1, w2, b2, w3_row, b3)


def kernel(user_ids, item_ids, user_table, item_table, W1, b1, W2, b2, W3, b3):
    uid = user_ids.astype(jnp.int32)
    iid = item_ids.astype(jnp.int32)
    upair = jnp.where(uid >= HALF, uid - HALF, uid).reshape(NW, NCHUNK, CHUNK)
    ipair = jnp.where(iid >= HALF, iid - HALF, iid).reshape(NW, NCHUNK, CHUNK)
    upar = (uid >= HALF).astype(jnp.int32).reshape(BATCH, 1)
    ipar = (iid >= HALF).astype(jnp.int32).reshape(BATCH, 1)
    utab2 = _relayout(user_table.T)
    u = _sc_gather(upair, utab2)
    itab2 = _relayout(item_table.T)
    i = _sc_gather(ipair, itab2)
    return _mlp(u, i, upar, ipar, W1[:EMBED], W1[EMBED:], b1.reshape(1, 128),
                W2, b2.reshape(1, 64), W3.reshape(1, 64), b3.reshape(1, 1))


# speedup vs baseline: 2.7877x; 1.1741x over previous
import functools

import jax
import jax.numpy as jnp
from jax import lax
from jax.experimental import pallas as pl
from jax.experimental.pallas import tpu as pltpu
from jax.experimental.pallas import tpu_sc as plsc

BATCH = 16384
EMBED = 64
NROWS = 1000000
NPAIR = NROWS // 2
NC = 2
NS = 16
NW = NC * NS
B_PER_W = BATCH // NW
CHUNK = 128
NCHUNK = B_PER_W // CHUNK
PAIR = 2 * EMBED
RELAYOUT_C = 16384
RELAYOUT_G = 31
HALF = RELAYOUT_G * RELAYOUT_C


def _relayout_body(lo_ref, hi_ref, eye_ref, out_ref):
    dn = (((0,), (0,)), ((), ()))
    lo = lo_ref[...].astype(jnp.bfloat16)
    hi = hi_ref[...].astype(jnp.bfloat16)
    eye = eye_ref[...]
    t_lo = lax.dot_general(lo, eye, dn, preferred_element_type=jnp.float32)
    t_hi = lax.dot_general(hi, eye, dn, preferred_element_type=jnp.float32)
    out_ref[...] = jnp.concatenate([t_lo, t_hi], axis=1)


def _relayout(tabT):
    return pl.pallas_call(
        _relayout_body,
        grid=(RELAYOUT_G,),
        in_specs=[
            pl.BlockSpec((EMBED, RELAYOUT_C), lambda c: (0, c)),
            pl.BlockSpec(
                (EMBED, RELAYOUT_C),
                lambda c: (0, jnp.minimum(c + RELAYOUT_G, NROWS // RELAYOUT_C))),
            pl.BlockSpec((EMBED, EMBED), lambda c: (0, 0)),
        ],
        out_specs=pl.BlockSpec((RELAYOUT_C, PAIR), lambda c: (c, 0)),
        out_shape=jax.ShapeDtypeStruct((HALF, PAIR), jnp.float32),
        compiler_params=pltpu.CompilerParams(fuse_transposed_lhs_in_matmul=True),
    )(tabT, tabT, jnp.eye(EMBED, dtype=jnp.bfloat16))


def _sc_gather_body(id_hbm, tab_hbm, out_hbm, idx_v, rows_v, sem):
    wid = lax.axis_index("s") * NC + lax.axis_index("c")
    base = wid * B_PER_W
    pltpu.sync_copy(id_hbm.at[wid], idx_v)
    copies = []
    for j in range(NCHUNK):
        copies.append(pltpu.async_copy(
            tab_hbm.at[idx_v.at[j]], rows_v.at[pl.ds(j * CHUNK, CHUNK)], sem))
    for c in copies:
        c.wait()
    pltpu.sync_copy(rows_v, out_hbm.at[pl.ds(base, B_PER_W)])


_sc_gather = functools.partial(
    pl.kernel,
    out_type=jax.ShapeDtypeStruct((BATCH, PAIR), jnp.float32),
    mesh=plsc.VectorSubcoreMesh(core_axis_name="c", subcore_axis_name="s"),
    scratch_types=[
        pltpu.VMEM((NCHUNK, CHUNK), jnp.int32),
        pltpu.VMEM((B_PER_W, PAIR), jnp.float32),
        pltpu.SemaphoreType.DMA,
    ],
)(_sc_gather_body)


def _mlp_body(u_ref, i_ref, up_ref, ip_ref, w1u_ref, w1i_ref, b1_ref,
              w2_ref, b2_ref, w3_ref, b3_ref, out_ref):
    u = jnp.where(up_ref[...] > 0, u_ref[:, EMBED:], u_ref[:, :EMBED])
    i = jnp.where(ip_ref[...] > 0, i_ref[:, EMBED:], i_ref[:, :EMBED])
    h = jnp.dot(u, w1u_ref[...], preferred_element_type=jnp.float32)
    h = h + jnp.dot(i, w1i_ref[...], preferred_element_type=jnp.float32)
    h = jnp.maximum(h + b1_ref[...], 0.0)
    h2 = jnp.dot(h, w2_ref[...], preferred_element_type=jnp.float32)
    h2 = jnp.maximum(h2 + b2_ref[...], 0.0)
    out = jnp.sum(h2 * w3_ref[...], axis=1, keepdims=True) + b3_ref[...]
    out_ref[...] = out


def _mlp(u, i, upar, ipar, w1u, w1i, b1, w2, b2, w3_row, b3):
    blk = 2048
    return pl.pallas_call(
        _mlp_body,
        grid=(BATCH // blk,),
        in_specs=[
            pl.BlockSpec((blk, PAIR), lambda g: (g, 0)),
            pl.BlockSpec((blk, PAIR), lambda g: (g, 0)),
            pl.BlockSpec((blk, 1), lambda g: (g, 0)),
            pl.BlockSpec((blk, 1), lambda g: (g, 0)),
            pl.BlockSpec((EMBED, 128), lambda g: (0, 0)),
            pl.BlockSpec((EMBED, 128), lambda g: (0, 0)),
            pl.BlockSpec((1, 128), lambda g: (0, 0)),
            pl.BlockSpec((128, 64), lambda g: (0, 0)),
            pl.BlockSpec((1, 64), lambda g: (0, 0)),
            pl.BlockSpec((1, 64), lambda g: (0, 0)),
            pl.BlockSpec((1, 1), lambda g: (0, 0)),
        ],
        out_specs=pl.BlockSpec((blk, 1), lambda g: (g, 0)),
        out_shape=jax.ShapeDtypeStruct((BATCH, 1), jnp.float32),
    )(u, i, upar, ipar, w1u, w1i, b1, w2, b2, w3_row, b3)


def kernel(user_ids, item_ids, user_table, item_table, W1, b1, W2, b2, W3, b3):
    uid = user_ids.astype(jnp.int32)
    iid = item_ids.astype(jnp.int32)
    upair = jnp.where(uid >= HALF, uid - HALF, uid).reshape(NW, NCHUNK, CHUNK)
    ipair = jnp.where(iid >= HALF, iid - HALF, iid).reshape(NW, NCHUNK, CHUNK)
    upar = (uid >= HALF).astype(jnp.int32).reshape(BATCH, 1)
    ipar = (iid >= HALF).astype(jnp.int32).reshape(BATCH, 1)
    utab2 = _relayout(user_table.T)
    u = _sc_gather(upair, utab2)
    itab2 = _relayout(item_table.T)
    i = _sc_gather(ipair, itab2)
    return _mlp(u, i, upar, ipar, W1[:EMBED], W1[EMBED:], b1.reshape(1, 128),
                W2, b2.reshape(1, 64), W3.reshape(1, 64), b3.reshape(1, 1))


# speedup vs baseline: 3.2449x; 1.1640x over previous
import functools

import jax
import jax.numpy as jnp
from jax import lax
from jax.experimental import pallas as pl
from jax.experimental.pallas import tpu as pltpu
from jax.experimental.pallas import tpu_sc as plsc

BATCH = 16384
EMBED = 64
NROWS = 1000000
NPAIR = NROWS // 2
NC = 2
NS = 16
NW = NC * NS
B_PER_W = BATCH // NW
CHUNK = 128
NCHUNK = B_PER_W // CHUNK
PAIR = 2 * EMBED
RELAYOUT_C = 8192
RELAYOUT_G = 32
QUART = RELAYOUT_G * RELAYOUT_C
QBITS = 18


def _relayout_body(q0_ref, q1_ref, q2_ref, q3_ref, eye_ref, out_ref):
    dn = (((0,), (0,)), ((), ()))
    eye = eye_ref[...]
    t = [lax.dot_general(r[...].astype(jnp.bfloat16), eye, dn,
                         preferred_element_type=jnp.float32)
         for r in (q0_ref, q1_ref, q2_ref, q3_ref)]
    u = [lax.bitcast_convert_type(x, jnp.uint32) for x in t]
    left = lax.bitcast_convert_type(u[0] | (u[1] >> 16), jnp.float32)
    right = lax.bitcast_convert_type(u[2] | (u[3] >> 16), jnp.float32)
    out_ref[...] = jnp.concatenate([left, right], axis=1)


def _relayout(tabT):
    nmax = NROWS // RELAYOUT_C
    return pl.pallas_call(
        _relayout_body,
        grid=(RELAYOUT_G,),
        in_specs=[
            pl.BlockSpec((EMBED, RELAYOUT_C), lambda c: (0, c)),
            pl.BlockSpec(
                (EMBED, RELAYOUT_C),
                lambda c: (0, jnp.minimum(c + RELAYOUT_G, NROWS // RELAYOUT_C))),
            pl.BlockSpec(
                (EMBED, RELAYOUT_C),
                lambda c: (0, jnp.minimum(c + 2 * RELAYOUT_G, NROWS // RELAYOUT_C))),
            pl.BlockSpec(
                (EMBED, RELAYOUT_C),
                lambda c: (0, jnp.minimum(c + 3 * RELAYOUT_G, NROWS // RELAYOUT_C))),
            pl.BlockSpec((EMBED, EMBED), lambda c: (0, 0)),
        ],
        out_specs=pl.BlockSpec((RELAYOUT_C, PAIR), lambda c: (c, 0)),
        out_shape=jax.ShapeDtypeStruct((QUART, PAIR), jnp.float32),
        compiler_params=pltpu.CompilerParams(fuse_transposed_lhs_in_matmul=True),
    )(tabT, tabT, tabT, tabT, jnp.eye(EMBED, dtype=jnp.bfloat16))


def _sc_gather_body(id_hbm, tab_hbm, out_hbm, idx_v, rows_v, sem):
    wid = lax.axis_index("s") * NC + lax.axis_index("c")
    base = wid * B_PER_W
    pltpu.sync_copy(id_hbm.at[wid], idx_v)
    copies = []
    for j in range(NCHUNK):
        copies.append(pltpu.async_copy(
            tab_hbm.at[idx_v.at[j]], rows_v.at[pl.ds(j * CHUNK, CHUNK)], sem))
    for c in copies:
        c.wait()
    pltpu.sync_copy(rows_v, out_hbm.at[pl.ds(base, B_PER_W)])


_sc_gather = functools.partial(
    pl.kernel,
    out_type=jax.ShapeDtypeStruct((BATCH, PAIR), jnp.float32),
    mesh=plsc.VectorSubcoreMesh(core_axis_name="c", subcore_axis_name="s"),
    scratch_types=[
        pltpu.VMEM((NCHUNK, CHUNK), jnp.int32),
        pltpu.VMEM((B_PER_W, PAIR), jnp.float32),
        pltpu.SemaphoreType.DMA,
    ],
)(_sc_gather_body)


def _unpack(ref, half_ref, low_ref):
    w = jnp.where(half_ref[...] > 0, ref[:, EMBED:], ref[:, :EMBED])
    b = lax.bitcast_convert_type(w, jnp.uint32)
    hi = lax.bitcast_convert_type(b & jnp.uint32(0xFFFF0000), jnp.float32)
    lo = lax.bitcast_convert_type(b << 16, jnp.float32)
    return jnp.where(low_ref[...] > 0, lo, hi)


def _mlp_body(u_ref, i_ref, uh_ref, ul_ref, ih_ref, il_ref,
              w1u_ref, w1i_ref, b1_ref,
              w2_ref, b2_ref, w3_ref, b3_ref, out_ref):
    u = _unpack(u_ref, uh_ref, ul_ref)
    i = _unpack(i_ref, ih_ref, il_ref)
    h = jnp.dot(u, w1u_ref[...], preferred_element_type=jnp.float32)
    h = h + jnp.dot(i, w1i_ref[...], preferred_element_type=jnp.float32)
    h = jnp.maximum(h + b1_ref[...], 0.0)
    h2 = jnp.dot(h, w2_ref[...], preferred_element_type=jnp.float32)
    h2 = jnp.maximum(h2 + b2_ref[...], 0.0)
    out = jnp.sum(h2 * w3_ref[...], axis=1, keepdims=True) + b3_ref[...]
    out_ref[...] = out


def _mlp(u, i, uh, ul, ih, il, w1u, w1i, b1, w2, b2, w3_row, b3):
    blk = 2048
    return pl.pallas_call(
        _mlp_body,
        grid=(BATCH // blk,),
        in_specs=[
            pl.BlockSpec((blk, PAIR), lambda g: (g, 0)),
            pl.BlockSpec((blk, PAIR), lambda g: (g, 0)),
            pl.BlockSpec((blk, 1), lambda g: (g, 0)),
            pl.BlockSpec((blk, 1), lambda g: (g, 0)),
            pl.BlockSpec((blk, 1), lambda g: (g, 0)),
            pl.BlockSpec((blk, 1), lambda g: (g, 0)),
            pl.BlockSpec((EMBED, 128), lambda g: (0, 0)),
            pl.BlockSpec((EMBED, 128), lambda g: (0, 0)),
            pl.BlockSpec((1, 128), lambda g: (0, 0)),
            pl.BlockSpec((128, 64), lambda g: (0, 0)),
            pl.BlockSpec((1, 64), lambda g: (0, 0)),
            pl.BlockSpec((1, 64), lambda g: (0, 0)),
            pl.BlockSpec((1, 1), lambda g: (0, 0)),
        ],
        out_specs=pl.BlockSpec((blk, 1), lambda g: (g, 0)),
        out_shape=jax.ShapeDtypeStruct((BATCH, 1), jnp.float32),
    )(u, i, uh, ul, ih, il, w1u, w1i, b1, w2, b2, w3_row, b3)


def kernel(user_ids, item_ids, user_table, item_table, W1, b1, W2, b2, W3, b3):
    uid = user_ids.astype(jnp.int32)
    iid = item_ids.astype(jnp.int32)
    uq = (uid & (QUART - 1)).reshape(NW, NCHUNK, CHUNK)
    iq = (iid & (QUART - 1)).reshape(NW, NCHUNK, CHUNK)
    uh = ((uid >> (QBITS + 1)) & 1).reshape(BATCH, 1)
    ul = ((uid >> QBITS) & 1).reshape(BATCH, 1)
    ih = ((iid >> (QBITS + 1)) & 1).reshape(BATCH, 1)
    il = ((iid >> QBITS) & 1).reshape(BATCH, 1)
    utab2 = _relayout(user_table.T)
    u = _sc_gather(uq, utab2)
    itab2 = _relayout(item_table.T)
    i = _sc_gather(iq, itab2)
    return _mlp(u, i, uh, ul, ih, il, W1[:EMBED], W1[EMBED:], b1.reshape(1, 128),
                W2, b2.reshape(1, 64), W3.reshape(1, 64), b3.reshape(1, 1))
